# baseline (speedup 1.0000x reference)
import math

import jax
import jax.numpy as jnp
from jax import lax
from jax.experimental import pallas as pl
from jax.experimental.pallas import tpu as pltpu
from jax.experimental.pallas import tpu_sc as plsc

_B = 64
_T = 45
_H = 256
_D = 23040
_N = 1024
_F32 = jnp.float32

_PAD_ROWS = 9216
_RPW = 288
_CHUNK = 96


def _lstm(x, Wih, Whh, bih, bhh):
    B = x.shape[0]

    def step(carry, xt):
        h, c = carry
        gates = xt @ Wih.T + h @ Whh.T + bih + bhh
        i, f, g, o = jnp.split(gates, 4, axis=-1)
        i = jax.nn.sigmoid(i)
        f = jax.nn.sigmoid(f)
        g = jnp.tanh(g)
        o = jax.nn.sigmoid(o)
        c = f * c + i * g
        h = o * jnp.tanh(c)
        return (h, c), h

    h0 = jnp.zeros((B, _H), dtype=x.dtype)
    c0 = jnp.zeros((B, _H), dtype=x.dtype)
    _, hs = jax.lax.scan(step, (h0, c0), jnp.transpose(x, (1, 0, 2)))
    return jnp.transpose(hs, (1, 0, 2))


def _softmax_body(s_ref, o_ref):
    s = s_ref[...]
    m = jnp.max(s, axis=1, keepdims=True)
    p = jnp.exp(s - m)
    o_ref[...] = p / jnp.sum(p, axis=1, keepdims=True)


def _topk_body(s_ref, idx_ref):
    x = s_ref[...]
    col = lax.broadcasted_iota(jnp.int32, (_B, _N), 1)
    cols = []
    for _ in range(3):
        m = jnp.max(x, axis=1, keepdims=True)
        am = jnp.min(jnp.where(x == m, col, _N), axis=1, keepdims=True)
        cols.append(am)
        x = jnp.where(col == am, -jnp.inf, x)
    idx_ref[...] = jnp.concatenate(cols, axis=1)


def _sc_gather_body(lab_ref, idx_ref, out_ref, idx_v, rows_v, sem):
    wid = lax.axis_index("s") * 2 + lax.axis_index("c")
    base = wid * _RPW
    for ch in range(_RPW // _CHUNK):
        off = base + ch * _CHUNK
        pltpu.sync_copy(idx_ref.at[pl.ds(off, _CHUNK)], idx_v)
        pltpu.async_copy(lab_ref.at[idx_v], rows_v, sem).wait()
        pltpu.sync_copy(rows_v, out_ref.at[pl.ds(off, _CHUNK)])


def _gather_rows(lab2, flat_idx):
    f = pl.kernel(
        _sc_gather_body,
        out_type=jax.ShapeDtypeStruct((_PAD_ROWS, 512), _F32),
        mesh=plsc.VectorSubcoreMesh(core_axis_name="c", subcore_axis_name="s"),
        scratch_types=[
            pltpu.VMEM((_CHUNK,), jnp.int32),
            pltpu.VMEM((_CHUNK, 512), _F32),
            pltpu.SemaphoreType.DMA,
        ],
    )
    return f(lab2, flat_idx)


def kernel(src, data, label, W1, b1, Wih_f, Whh_f, bih_f, bhh_f,
           Wih_b, Whh_b, bih_b, bhh_b):
    B, S, T, Fd = src.shape
    x = jnp.transpose(src, (0, 2, 1, 3)).reshape(B, T, S * Fd)
    x = jnp.transpose(x, (0, 2, 1))
    x = x @ W1.T + b1
    x = jnp.transpose(x, (0, 2, 1))
    hf = _lstm(x, Wih_f, Whh_f, bih_f, bhh_f)
    hb = jnp.flip(_lstm(jnp.flip(x, 1), Wih_b, Whh_b, bih_b, bhh_b), 1)
    e_imu = jnp.concatenate([hf, hb], axis=-1)

    q = e_imu.reshape(B, -1)
    kmat = data.reshape(_N, -1)
    score = (q @ kmat.T) / math.sqrt(512)

    attn = pl.pallas_call(
        _softmax_body,
        out_shape=jax.ShapeDtypeStruct((_B, _N), _F32),
    )(score)
    e2 = (attn @ kmat)

    lab2 = label.reshape(_N, _D)
    q2 = e2
    qn = jnp.maximum(jnp.linalg.norm(q2, axis=-1, keepdims=True), 1e-8)
    ln = jnp.maximum(jnp.linalg.norm(lab2, axis=-1, keepdims=True), 1e-8)
    sims = (q2 / qn) @ (lab2 / ln).T
    idx = pl.pallas_call(
        _topk_body,
        out_shape=jax.ShapeDtypeStruct((_B, 3), jnp.int32),
    )(sims)

    flat = (idx.reshape(_B * 3, 1) * _T
            + jnp.arange(_T, dtype=jnp.int32).reshape(1, _T)).reshape(-1)
    flat = jnp.concatenate(
        [flat, jnp.zeros((_PAD_ROWS - _B * 3 * _T,), jnp.int32)])
    rows = _gather_rows(label.reshape(_N * _T, 512), flat)
    labels = rows[:_B * 3 * _T].reshape(_B, 3 * _T, 512)
    return e2.reshape(B, _T, 512), labels
